# speedup vs baseline: 1.2700x; 1.2700x over previous
import functools

import jax
import jax.numpy as jnp
from jax import lax
from jax.experimental import pallas as pl
from jax.experimental.pallas import tpu as pltpu
from jax.experimental.pallas import tpu_sc as plsc

EMBED_DIM = 128
SEQLEN = 2048
BATCH = 4
TOTAL = BATCH * SEQLEN
NUM_CORES = 2
NUM_SUBCORES = 16
NUM_WORKERS = NUM_CORES * NUM_SUBCORES
ROWS_PER_W = TOTAL // NUM_WORKERS
CHUNK = 128
NCHUNK = ROWS_PER_W // CHUNK
LANES = 16


@functools.partial(
    pl.kernel,
    mesh=plsc.VectorSubcoreMesh(core_axis_name="c", subcore_axis_name="s"),
    out_type=jax.ShapeDtypeStruct((TOTAL, EMBED_DIM), jnp.float32),
    scratch_types=[
        pltpu.VMEM((NCHUNK, CHUNK), jnp.int32),
        pltpu.VMEM((ROWS_PER_W, EMBED_DIM), jnp.float32),
        pltpu.VMEM((ROWS_PER_W, EMBED_DIM), jnp.float32),
        pltpu.SemaphoreType.DMA,
    ],
)
def _embed_kernel(ids_hbm, word_hbm, pos_hbm, out_hbm, idx_v, rows_v, pos_v, sem):
    wid = lax.axis_index("s") * NUM_CORES + lax.axis_index("c")
    base = wid * ROWS_PER_W
    pbase = lax.rem(base, SEQLEN)

    pltpu.sync_copy(ids_hbm.at[wid], idx_v)

    copies = [
        pltpu.async_copy(
            word_hbm.at[idx_v.at[j]],
            rows_v.at[pl.ds(j * CHUNK, CHUNK)],
            sem,
        )
        for j in range(NCHUNK)
    ]
    pltpu.sync_copy(pos_hbm.at[pl.ds(pbase, ROWS_PER_W)], pos_v)
    for cp in copies:
        cp.wait()

    def add_row(r, carry):
        for c in range(EMBED_DIM // LANES):
            sl = pl.ds(c * LANES, LANES)
            rows_v[r, sl] = rows_v[r, sl] + pos_v[r, sl]
        return carry

    lax.fori_loop(0, ROWS_PER_W, add_row, 0)

    pltpu.sync_copy(rows_v, out_hbm.at[pl.ds(base, ROWS_PER_W)])


def kernel(input_ids, word_table, pos_table):
    ids = input_ids.reshape(NUM_WORKERS, NCHUNK, CHUNK).astype(jnp.int32)
    out = _embed_kernel(ids, word_table, pos_table)
    return out.reshape(BATCH, SEQLEN, EMBED_DIM)


# speedup vs baseline: 1.3494x; 1.0626x over previous
import functools

import jax
import jax.numpy as jnp
from jax import lax
from jax.experimental import pallas as pl
from jax.experimental.pallas import tpu as pltpu
from jax.experimental.pallas import tpu_sc as plsc

EMBED_DIM = 128
SEQLEN = 2048
BATCH = 4
TOTAL = BATCH * SEQLEN
NUM_CORES = 2
NUM_SUBCORES = 16
NUM_WORKERS = NUM_CORES * NUM_SUBCORES
ROWS_PER_W = TOTAL // NUM_WORKERS
CHUNK = 128
NCHUNK = ROWS_PER_W // CHUNK
LANES = 16


@functools.partial(
    pl.kernel,
    mesh=plsc.VectorSubcoreMesh(core_axis_name="c", subcore_axis_name="s"),
    out_type=jax.ShapeDtypeStruct((TOTAL, EMBED_DIM), jnp.float32),
    scratch_types=[
        pltpu.VMEM((NCHUNK, CHUNK), jnp.int32),
        pltpu.VMEM((ROWS_PER_W, EMBED_DIM), jnp.float32),
        pltpu.SemaphoreType.DMA,
    ],
)
def _embed_kernel(ids_hbm, word_hbm, pos_hbm, out_hbm, idx_v, rows_v, sem):
    wid = lax.axis_index("s") * NUM_CORES + lax.axis_index("c")
    base = wid * ROWS_PER_W
    pbase = lax.rem(base, SEQLEN)

    pltpu.sync_copy(ids_hbm.at[wid], idx_v)
    pltpu.sync_copy(pos_hbm.at[pl.ds(pbase, ROWS_PER_W)], rows_v)

    copies = [
        pltpu.async_copy(
            word_hbm.at[idx_v.at[j]],
            rows_v.at[pl.ds(j * CHUNK, CHUNK)],
            sem,
            add=True,
        )
        for j in range(NCHUNK)
    ]
    for cp in copies:
        cp.wait()

    pltpu.sync_copy(rows_v, out_hbm.at[pl.ds(base, ROWS_PER_W)])


def kernel(input_ids, word_table, pos_table):
    ids = input_ids.reshape(NUM_WORKERS, NCHUNK, CHUNK).astype(jnp.int32)
    out = _embed_kernel(ids, word_table, pos_table)
    return out.reshape(BATCH, SEQLEN, EMBED_DIM)
